# baseline (speedup 1.0000x reference)
import jax
import jax.numpy as jnp
from jax.experimental import pallas as pl
from jax.experimental.pallas import tpu as pltpu

_TPAD = 1024


def _loss_body(melt_ref, melp_ref, post_ref, cmask_ref, cmask_full_ref,
               pt_ref, pp_ref, et_ref, ep_ref, ldp_ref, dur_ref,
               total_ref, mel_ref, post_ref_o, pitch_ref, energy_ref, durl_ref,
               acc_ref):
    step = pl.program_id(0)
    nsteps = pl.num_programs(0)
    T_mel = melt_ref.shape[2]
    T_text = pt_ref.shape[1]

    @pl.when(step == 0)
    def _():
        tm = cmask_full_ref[:, _TPAD:_TPAD + T_text]
        pe = (pp_ref[...] - pt_ref[...]) ** 2
        ee = (ep_ref[...] - et_ref[...]) ** 2
        ldt = jnp.log(dur_ref[...].astype(jnp.float32) + 1.0)
        de = (ldp_ref[...] - ldt) ** 2
        acc_ref[0] = 0.0
        acc_ref[1] = 0.0
        acc_ref[2] = 0.0
        acc_ref[3] = jnp.sum(pe * tm)
        acc_ref[4] = jnp.sum(ee * tm)
        acc_ref[5] = jnp.sum(de * tm)
        acc_ref[6] = jnp.sum(tm)

    t = melt_ref[...]
    BBk = melt_ref.shape[0]
    sel = jnp.where(
        jax.lax.broadcasted_iota(jnp.int32, (BBk, cmask_ref.shape[0]), 0)
        + step * BBk
        == jax.lax.broadcasted_iota(jnp.int32, (BBk, cmask_ref.shape[0]), 1),
        1.0, 0.0)
    m = jnp.dot(sel, cmask_ref[:, 0:T_mel], preferred_element_type=jnp.float32)
    mb = m[:, None, :]
    d1 = jnp.abs(melp_ref[...] - t) * mb
    d2 = jnp.abs(post_ref[...] - t) * mb
    acc_ref[0] += jnp.sum(d1)
    acc_ref[1] += jnp.sum(d2)
    acc_ref[2] += jnp.sum(m)

    @pl.when(step == nsteps - 1)
    def _():
        n_mels_f = jnp.float32(melt_ref.shape[1])
        denom = acc_ref[2] * n_mels_f
        mel_loss = acc_ref[0] / denom
        postnet_mel_loss = acc_ref[1] / denom
        tsum = acc_ref[6]
        pitch_loss = acc_ref[3] / tsum
        energy_loss = acc_ref[4] / tsum
        duration_loss = acc_ref[5] / tsum
        mel_ref[0] = mel_loss
        post_ref_o[0] = postnet_mel_loss
        pitch_ref[0] = pitch_loss
        energy_ref[0] = energy_loss
        durl_ref[0] = duration_loss
        total_ref[0] = (mel_loss + postnet_mel_loss + duration_loss
                        + pitch_loss + energy_loss)


def kernel(mel_targets, pitch_targets, energy_targets, duration_targets,
           mel_predictions, postnet_mel_predictions, pitch_predictions,
           energy_predictions, log_duration_predictions, text_masks, mel_masks):
    B, T_mel, n_mels = mel_targets.shape
    T_text = pitch_targets.shape[1]

    mt = jnp.swapaxes(mel_targets, 1, 2)
    mp = jnp.swapaxes(mel_predictions, 1, 2)
    po = jnp.swapaxes(postnet_mel_predictions, 1, 2)

    cmask = jnp.concatenate(
        [jnp.logical_not(mel_masks),
         jnp.zeros((B, _TPAD - T_mel), jnp.bool_),
         jnp.logical_not(text_masks)], axis=1).astype(jnp.float32)

    BB = 4
    W = _TPAD + T_text
    scalar = jax.ShapeDtypeStruct((1,), jnp.float32)
    outs = pl.pallas_call(
        _loss_body,
        grid=(B // BB,),
        in_specs=[
            pl.BlockSpec((BB, n_mels, T_mel), lambda b: (b, 0, 0)),
            pl.BlockSpec((BB, n_mels, T_mel), lambda b: (b, 0, 0)),
            pl.BlockSpec((BB, n_mels, T_mel), lambda b: (b, 0, 0)),
            pl.BlockSpec((B, W), lambda b: (0, 0)),
            pl.BlockSpec((B, W), lambda b: (0, 0)),
            pl.BlockSpec((B, T_text), lambda b: (0, 0)),
            pl.BlockSpec((B, T_text), lambda b: (0, 0)),
            pl.BlockSpec((B, T_text), lambda b: (0, 0)),
            pl.BlockSpec((B, T_text), lambda b: (0, 0)),
            pl.BlockSpec((B, T_text), lambda b: (0, 0)),
            pl.BlockSpec((B, T_text), lambda b: (0, 0)),
        ],
        out_specs=[pl.BlockSpec(memory_space=pltpu.SMEM)] * 6,
        out_shape=[scalar] * 6,
        scratch_shapes=[pltpu.SMEM((8,), jnp.float32)],
    )(mt, mp, po, cmask, cmask,
      pitch_targets, pitch_predictions, energy_targets, energy_predictions,
      log_duration_predictions, duration_targets)

    total_loss, mel_loss, postnet_mel_loss, pitch_loss, energy_loss, \
        duration_loss = (o.reshape(()) for o in outs)
    return (total_loss, mel_loss, postnet_mel_loss, pitch_loss,
            energy_loss, duration_loss)


# speedup vs baseline: 1.1211x; 1.1211x over previous
import jax
import jax.numpy as jnp
from jax.experimental import pallas as pl
from jax.experimental.pallas import tpu as pltpu

_TPAD = 1024


def _loss_body(melt_ref, melp_ref, post_ref, cmask_ref, cmask_full_ref,
               pt_ref, pp_ref, et_ref, ep_ref, ldp_ref, dur_ref,
               total_ref, mel_ref, post_ref_o, pitch_ref, energy_ref, durl_ref,
               acc_ref):
    step = pl.program_id(0)
    nsteps = pl.num_programs(0)
    T_mel = melt_ref.shape[2]
    T_text = pt_ref.shape[1]

    @pl.when(step == 0)
    def _():
        tm = cmask_full_ref[:, _TPAD:_TPAD + T_text]
        pe = (pp_ref[...] - pt_ref[...]) ** 2
        ee = (ep_ref[...] - et_ref[...]) ** 2
        ldt = jnp.log(dur_ref[...].astype(jnp.float32) + 1.0)
        de = (ldp_ref[...] - ldt) ** 2
        acc_ref[0] = 0.0
        acc_ref[1] = 0.0
        acc_ref[2] = 0.0
        acc_ref[3] = jnp.sum(pe * tm)
        acc_ref[4] = jnp.sum(ee * tm)
        acc_ref[5] = jnp.sum(de * tm)
        acc_ref[6] = jnp.sum(tm)

    t = melt_ref[...]
    m = cmask_ref[:, 0:T_mel]
    mb = m[:, None, :]
    d1 = jnp.abs(melp_ref[...] - t) * mb
    d2 = jnp.abs(post_ref[...] - t) * mb
    acc_ref[0] += jnp.sum(d1)
    acc_ref[1] += jnp.sum(d2)
    acc_ref[2] += jnp.sum(m)

    @pl.when(step == nsteps - 1)
    def _():
        n_mels_f = jnp.float32(melt_ref.shape[1])
        denom = acc_ref[2] * n_mels_f
        mel_loss = acc_ref[0] / denom
        postnet_mel_loss = acc_ref[1] / denom
        tsum = acc_ref[6]
        pitch_loss = acc_ref[3] / tsum
        energy_loss = acc_ref[4] / tsum
        duration_loss = acc_ref[5] / tsum
        mel_ref[0] = mel_loss
        post_ref_o[0] = postnet_mel_loss
        pitch_ref[0] = pitch_loss
        energy_ref[0] = energy_loss
        durl_ref[0] = duration_loss
        total_ref[0] = (mel_loss + postnet_mel_loss + duration_loss
                        + pitch_loss + energy_loss)


def kernel(mel_targets, pitch_targets, energy_targets, duration_targets,
           mel_predictions, postnet_mel_predictions, pitch_predictions,
           energy_predictions, log_duration_predictions, text_masks, mel_masks):
    B, T_mel, n_mels = mel_targets.shape
    T_text = pitch_targets.shape[1]

    mt = jnp.swapaxes(mel_targets, 1, 2)
    mp = jnp.swapaxes(mel_predictions, 1, 2)
    po = jnp.swapaxes(postnet_mel_predictions, 1, 2)

    cmask = jnp.concatenate(
        [jnp.logical_not(mel_masks),
         jnp.zeros((B, _TPAD - T_mel), jnp.bool_),
         jnp.logical_not(text_masks)], axis=1).astype(jnp.float32)

    BB = 8
    W = _TPAD + T_text
    scalar = jax.ShapeDtypeStruct((1,), jnp.float32)
    outs = pl.pallas_call(
        _loss_body,
        grid=(B // BB,),
        in_specs=[
            pl.BlockSpec((BB, n_mels, T_mel), lambda b: (b, 0, 0)),
            pl.BlockSpec((BB, n_mels, T_mel), lambda b: (b, 0, 0)),
            pl.BlockSpec((BB, n_mels, T_mel), lambda b: (b, 0, 0)),
            pl.BlockSpec((BB, W), lambda b: (b, 0)),
            pl.BlockSpec((B, W), lambda b: (0, 0)),
            pl.BlockSpec((B, T_text), lambda b: (0, 0)),
            pl.BlockSpec((B, T_text), lambda b: (0, 0)),
            pl.BlockSpec((B, T_text), lambda b: (0, 0)),
            pl.BlockSpec((B, T_text), lambda b: (0, 0)),
            pl.BlockSpec((B, T_text), lambda b: (0, 0)),
            pl.BlockSpec((B, T_text), lambda b: (0, 0)),
        ],
        out_specs=[pl.BlockSpec(memory_space=pltpu.SMEM)] * 6,
        out_shape=[scalar] * 6,
        scratch_shapes=[pltpu.SMEM((8,), jnp.float32)],
    )(mt, mp, po, cmask, cmask,
      pitch_targets, pitch_predictions, energy_targets, energy_predictions,
      log_duration_predictions, duration_targets)

    total_loss, mel_loss, postnet_mel_loss, pitch_loss, energy_loss, \
        duration_loss = (o.reshape(()) for o in outs)
    return (total_loss, mel_loss, postnet_mel_loss, pitch_loss,
            energy_loss, duration_loss)
